# baseline (speedup 1.0000x reference)
import jax
import jax.numpy as jnp
from jax.experimental import pallas as pl


_ROWS_PER_BLOCK = 32


def _body(x_ref, o_ref, idx_ref):
    x = x_ref[...]
    o_ref[...] = x
    idx_ref[...] = jnp.argmin(x, axis=1, keepdims=True).astype(jnp.int32)


def kernel(inputs):
    m, n = inputs.shape
    rb = _ROWS_PER_BLOCK
    grid = (m // rb,)
    out, idx = pl.pallas_call(
        _body,
        grid=grid,
        in_specs=[pl.BlockSpec((rb, n), lambda i: (i, 0))],
        out_specs=[
            pl.BlockSpec((rb, n), lambda i: (i, 0)),
            pl.BlockSpec((rb, 1), lambda i: (i, 0)),
        ],
        out_shape=[
            jax.ShapeDtypeStruct((m, n), inputs.dtype),
            jax.ShapeDtypeStruct((m, 1), jnp.int32),
        ],
    )(inputs)
    del idx
    return out


# speedup vs baseline: 1.0717x; 1.0717x over previous
import jax
import jax.numpy as jnp
from jax.experimental import pallas as pl


_ROWS_PER_BLOCK = 32


def _body(x_ref, o_ref, idx_ref):
    x = x_ref[...]
    o_ref[...] = x
    idx_ref[...] = jnp.min(x, axis=1, keepdims=True).astype(jnp.float32)


def kernel(inputs):
    m, n = inputs.shape
    rb = _ROWS_PER_BLOCK
    grid = (m // rb,)
    out, idx = pl.pallas_call(
        _body,
        grid=grid,
        in_specs=[pl.BlockSpec((rb, n), lambda i: (i, 0))],
        out_specs=[
            pl.BlockSpec((rb, n), lambda i: (i, 0)),
            pl.BlockSpec((rb, 1), lambda i: (i, 0)),
        ],
        out_shape=[
            jax.ShapeDtypeStruct((m, n), inputs.dtype),
            jax.ShapeDtypeStruct((m, 1), jnp.float32),
        ],
    )(inputs)
    del idx
    return out


# speedup vs baseline: 1.1408x; 1.0645x over previous
import functools

import jax
import jax.numpy as jnp
from jax import lax
from jax.experimental import pallas as pl
from jax.experimental.pallas import tpu as pltpu
from jax.experimental.pallas import tpu_sc as plsc


_ROWS_PER_BLOCK = 32

_NC = 2
_NS = 16
_LANES = 16
_BIG = 2**30


def _copy_body(x_ref, o_ref):
    o_ref[...] = x_ref[...]


def _tc_copy(inputs):
    m, n = inputs.shape
    rb = _ROWS_PER_BLOCK
    return pl.pallas_call(
        _copy_body,
        grid=(m // rb,),
        in_specs=[pl.BlockSpec((rb, n), lambda i: (i, 0))],
        out_specs=pl.BlockSpec((rb, n), lambda i: (i, 0)),
        out_shape=jax.ShapeDtypeStruct((m, n), inputs.dtype),
    )(inputs)


_GATHER_DNUMS = lax.GatherDimensionNumbers(
    offset_dims=(), collapsed_slice_dims=(0,), start_index_map=(0,))


def _permute(v, idx):
    return lax.gather(v, idx[:, None], _GATHER_DNUMS, slice_sizes=(1,),
                      mode=lax.GatherScatterMode.PROMISE_IN_BOUNDS)


def _lane_min(v):
    lanes = lax.iota(jnp.int32, _LANES)
    for k in (8, 4, 2, 1):
        v = jnp.minimum(v, _permute(v, lanes ^ k))
    return v


def _row_argmin(rowbuf, n):
    nvec = n // _LANES
    lanes = lax.iota(jnp.int32, _LANES)
    inf = jnp.full((_LANES,), jnp.inf, jnp.float32)
    zero = jnp.zeros((_LANES,), jnp.int32)

    def body(j, carry):
        runmin, runj = carry
        v = rowbuf[pl.ds(j * _LANES, _LANES)]
        lt = v < runmin
        runj = jnp.where(lt, jnp.full((_LANES,), 1, jnp.int32) * j, runj)
        runmin = jnp.minimum(runmin, v)
        return runmin, runj

    runmin, runj = lax.fori_loop(0, nvec, body, (inf, zero), unroll=8)
    m = _lane_min(runmin)
    cand = jnp.where(runmin == m, runj * _LANES + lanes, _BIG)
    return _lane_min(cand)


def _sc_argmin(inputs):
    m, n = inputs.shape
    nworkers = _NC * _NS
    rows_per_w = m // nworkers
    mesh = plsc.VectorSubcoreMesh(core_axis_name="c", subcore_axis_name="s")

    @functools.partial(
        pl.kernel,
        out_type=jax.ShapeDtypeStruct((nworkers, _LANES), jnp.int32),
        mesh=mesh,
        scratch_types=[
            pltpu.VMEM((n,), jnp.float32),
            pltpu.VMEM((n,), jnp.float32),
            pltpu.VMEM((_LANES,), jnp.int32),
            pltpu.SemaphoreType.DMA,
            pltpu.SemaphoreType.DMA,
        ],
    )
    def k(x_hbm, out_hbm, buf0, buf1, idx_v, sem0, sem1):
        wid = lax.axis_index("s") * _NC + lax.axis_index("c")
        base = wid * rows_per_w
        bufs = (buf0, buf1)
        sems = (sem0, sem1)
        lanes = lax.iota(jnp.int32, _LANES)

        copies = [None, None]
        copies[0] = pltpu.make_async_copy(x_hbm.at[base], buf0, sem0)
        copies[0].start()
        res = jnp.zeros((_LANES,), jnp.int32)
        for r in range(rows_per_w):
            cur = r % 2
            copies[cur].wait()
            if r + 1 < rows_per_w:
                nxt = (r + 1) % 2
                copies[nxt] = pltpu.make_async_copy(
                    x_hbm.at[base + r + 1], bufs[nxt], sems[nxt])
                copies[nxt].start()
            idx = _row_argmin(bufs[cur], n)
            res = jnp.where(lanes == r, idx, res)
        idx_v[...] = res
        pltpu.sync_copy(idx_v, out_hbm.at[wid])

    return k(inputs)


def kernel(inputs):
    out = _tc_copy(inputs)
    idx = _sc_argmin(inputs)
    out, _ = lax.optimization_barrier((out, idx))
    return out
